# baseline (speedup 1.0000x reference)
import functools

import jax
import jax.numpy as jnp
from jax import lax
from jax.experimental import pallas as pl
from jax.experimental.pallas import tpu as pltpu
from jax.experimental.pallas import tpu_sc as plsc

BETA = 0.25
N_TOK = 8192
DIM = 256
ROWS = 8192
RB = 512
NI = ROWS // RB

_SC_NC = 2
_SC_NS = 16
_SC_NW = _SC_NC * _SC_NS
_BPW = ROWS // _SC_NW


CW = 2048
NCH = N_TOK // CW


def _vq_body(z_ref, wt_ref, idx_ref, loss_ref, wn_ref, colf_ref):
    i = pl.program_id(0)

    @pl.when(i == 0)
    def _():
        wt = wt_ref[...]
        wn_ref[...] = jnp.sum(wt * wt, axis=0, keepdims=True)
        colf_ref[...] = lax.broadcasted_iota(
            jnp.int32, (1, N_TOK), 1).astype(jnp.float32)
        loss_ref[...] = jnp.zeros((1, 1), jnp.float32)

    zb = z_ref[...]
    zn = jnp.sum(zb * zb, axis=1, keepdims=True)
    zb2 = zb * (-2.0)
    best_m = best_i = None
    for c in range(NCH):
        sl = pl.ds(c * CW, CW)
        s2 = lax.dot_general(zb2, wt_ref[:, sl], (((1,), (0,)), ((), ())),
                             preferred_element_type=jnp.float32)
        d = (zn + wn_ref[:, sl]) + s2
        mc = jnp.min(d, axis=1, keepdims=True)
        ic = jnp.min(jnp.where(d == mc, colf_ref[:, sl], float(N_TOK)),
                     axis=1, keepdims=True)
        if c == 0:
            best_m, best_i = mc, ic
        else:
            take = mc < best_m
            best_i = jnp.where(take, ic, best_i)
            best_m = jnp.where(take, mc, best_m)
    idx_ref[0, 0, :] = best_i[:, 0].astype(jnp.int32)
    loss_ref[...] += jnp.sum(best_m).reshape(1, 1)

    @pl.when(i == NI - 1)
    def _():
        loss_ref[...] = loss_ref[...] * (BETA / float(ROWS * DIM))


_vq_call = pl.pallas_call(
    _vq_body,
    grid=(NI,),
    in_specs=[
        pl.BlockSpec((RB, DIM), lambda i: (i, 0)),
        pl.BlockSpec((DIM, N_TOK), lambda i: (0, 0)),
    ],
    out_specs=[
        pl.BlockSpec((1, 1, RB), lambda i: (i, 0, 0)),
        pl.BlockSpec((1, 1), lambda i: (0, 0)),
    ],
    out_shape=[
        jax.ShapeDtypeStruct((NI, 1, RB), jnp.int32),
        jax.ShapeDtypeStruct((1, 1), jnp.float32),
    ],
    scratch_shapes=[pltpu.VMEM((1, N_TOK), jnp.float32),
                    pltpu.VMEM((1, N_TOK), jnp.float32)],
)


@functools.cache
def _sc_gather_call():
    @functools.partial(
        pl.kernel,
        mesh=plsc.VectorSubcoreMesh(core_axis_name="c", subcore_axis_name="s"),
        out_type=jax.ShapeDtypeStruct((ROWS, DIM), jnp.float32),
        scratch_types=[
            pltpu.VMEM((_BPW,), jnp.int32),
            pltpu.VMEM((_BPW, DIM), jnp.float32),
            pltpu.SemaphoreType.DMA,
        ],
    )
    def _sc_gather(table_hbm, idx_hbm, out_hbm, idx_v, rows_v, sem):
        wid = lax.axis_index("s") * _SC_NC + lax.axis_index("c")
        base = wid * _BPW
        pltpu.sync_copy(idx_hbm.at[pl.ds(base, _BPW)], idx_v)
        pltpu.async_copy(table_hbm.at[idx_v], rows_v, sem).wait()
        pltpu.sync_copy(rows_v, out_hbm.at[pl.ds(base, _BPW)])

    return _sc_gather


def kernel(z, W):
    zp = jnp.transpose(z, (0, 2, 3, 1))
    z_flat = zp.reshape(ROWS, DIM)
    wt = W.T
    idx3, loss11 = _vq_call(z_flat, wt)
    idx = idx3.reshape(ROWS)
    z_q_rows = _sc_gather_call()(W, idx)
    z_q_out = jnp.transpose(z_q_rows.reshape(zp.shape), (0, 3, 1, 2))
    return (z_q_out, loss11[0, 0], idx)


# speedup vs baseline: 1.0693x; 1.0693x over previous
import functools

import jax
import jax.numpy as jnp
from jax import lax
from jax.experimental import pallas as pl
from jax.experimental.pallas import tpu as pltpu
from jax.experimental.pallas import tpu_sc as plsc

BETA = 0.25
N_TOK = 8192
DIM = 256
ROWS = 8192
RB = 1024
NI = ROWS // RB

_SC_NC = 2
_SC_NS = 16
_SC_NW = _SC_NC * _SC_NS
_BPW = ROWS // _SC_NW


CW = 1024
NCH = N_TOK // CW


def _vq_body(z_ref, wt_ref, idx_ref, loss_ref, wn_ref, colf_ref):
    i = pl.program_id(0)

    @pl.when(i == 0)
    def _():
        wt = wt_ref[...]
        wn_ref[...] = jnp.sum(wt * wt, axis=0, keepdims=True)
        colf_ref[...] = lax.broadcasted_iota(
            jnp.int32, (1, N_TOK), 1).astype(jnp.float32)
        loss_ref[...] = jnp.zeros((1, 1), jnp.float32)

    zb = z_ref[...]
    zn = jnp.sum(zb * zb, axis=1, keepdims=True)
    zb2 = zb * (-2.0)
    best_m = best_i = None
    for c in range(NCH):
        sl = pl.ds(c * CW, CW)
        s2 = lax.dot_general(zb2, wt_ref[:, sl], (((1,), (0,)), ((), ())),
                             preferred_element_type=jnp.float32)
        d = (zn + wn_ref[:, sl]) + s2
        mc = jnp.min(d, axis=1, keepdims=True)
        ic = jnp.min(jnp.where(d == mc, colf_ref[:, sl], float(N_TOK)),
                     axis=1, keepdims=True)
        if c == 0:
            best_m, best_i = mc, ic
        else:
            take = mc < best_m
            best_i = jnp.where(take, ic, best_i)
            best_m = jnp.where(take, mc, best_m)
    idx_ref[0, 0, :] = best_i[:, 0].astype(jnp.int32)
    loss_ref[...] += jnp.sum(best_m).reshape(1, 1)

    @pl.when(i == NI - 1)
    def _():
        loss_ref[...] = loss_ref[...] * (BETA / float(ROWS * DIM))


_vq_call = pl.pallas_call(
    _vq_body,
    grid=(NI,),
    in_specs=[
        pl.BlockSpec((RB, DIM), lambda i: (i, 0)),
        pl.BlockSpec((DIM, N_TOK), lambda i: (0, 0)),
    ],
    out_specs=[
        pl.BlockSpec((1, 1, RB), lambda i: (i, 0, 0)),
        pl.BlockSpec((1, 1), lambda i: (0, 0)),
    ],
    out_shape=[
        jax.ShapeDtypeStruct((NI, 1, RB), jnp.int32),
        jax.ShapeDtypeStruct((1, 1), jnp.float32),
    ],
    scratch_shapes=[pltpu.VMEM((1, N_TOK), jnp.float32),
                    pltpu.VMEM((1, N_TOK), jnp.float32)],
)


@functools.cache
def _sc_gather_call():
    @functools.partial(
        pl.kernel,
        mesh=plsc.VectorSubcoreMesh(core_axis_name="c", subcore_axis_name="s"),
        out_type=jax.ShapeDtypeStruct((ROWS, DIM), jnp.float32),
        scratch_types=[
            pltpu.VMEM((_BPW,), jnp.int32),
            pltpu.VMEM((_BPW, DIM), jnp.float32),
            pltpu.SemaphoreType.DMA,
        ],
    )
    def _sc_gather(table_hbm, idx_hbm, out_hbm, idx_v, rows_v, sem):
        wid = lax.axis_index("s") * _SC_NC + lax.axis_index("c")
        base = wid * _BPW
        pltpu.sync_copy(idx_hbm.at[pl.ds(base, _BPW)], idx_v)
        pltpu.async_copy(table_hbm.at[idx_v], rows_v, sem).wait()
        pltpu.sync_copy(rows_v, out_hbm.at[pl.ds(base, _BPW)])

    return _sc_gather


def kernel(z, W):
    zp = jnp.transpose(z, (0, 2, 3, 1))
    z_flat = zp.reshape(ROWS, DIM)
    wt = W.T
    idx3, loss11 = _vq_call(z_flat, wt)
    idx = idx3.reshape(ROWS)
    z_q_rows = _sc_gather_call()(W, idx)
    z_q_out = jnp.transpose(z_q_rows.reshape(zp.shape), (0, 3, 1, 2))
    return (z_q_out, loss11[0, 0], idx)


# speedup vs baseline: 1.0828x; 1.0126x over previous
import functools

import jax
import jax.numpy as jnp
from jax import lax
from jax.experimental import pallas as pl
from jax.experimental.pallas import tpu as pltpu
from jax.experimental.pallas import tpu_sc as plsc

BETA = 0.25
N_TOK = 8192
DIM = 256
ROWS = 8192
RB = 2048
NI = ROWS // RB

_SC_NC = 2
_SC_NS = 16
_SC_NW = _SC_NC * _SC_NS
_BPW = ROWS // _SC_NW


CW = 1024
NCH = N_TOK // CW


def _vq_body(z_ref, wt_ref, idx_ref, loss_ref, wn_ref, colf_ref):
    i = pl.program_id(0)

    @pl.when(i == 0)
    def _():
        wt = wt_ref[...]
        wn_ref[...] = jnp.sum(wt * wt, axis=0, keepdims=True)
        colf_ref[...] = lax.broadcasted_iota(
            jnp.int32, (1, N_TOK), 1).astype(jnp.float32)
        loss_ref[...] = jnp.zeros((1, 1), jnp.float32)

    zb = z_ref[...]
    zn = jnp.sum(zb * zb, axis=1, keepdims=True)
    zb2 = zb * (-2.0)
    best_m = best_i = None
    for c in range(NCH):
        sl = pl.ds(c * CW, CW)
        s2 = lax.dot_general(zb2, wt_ref[:, sl], (((1,), (0,)), ((), ())),
                             preferred_element_type=jnp.float32)
        d = (zn + wn_ref[:, sl]) + s2
        mc = jnp.min(d, axis=1, keepdims=True)
        ic = jnp.min(jnp.where(d == mc, colf_ref[:, sl], float(N_TOK)),
                     axis=1, keepdims=True)
        if c == 0:
            best_m, best_i = mc, ic
        else:
            take = mc < best_m
            best_i = jnp.where(take, ic, best_i)
            best_m = jnp.where(take, mc, best_m)
    idx_ref[0, 0, :] = best_i[:, 0].astype(jnp.int32)
    loss_ref[...] += jnp.sum(best_m).reshape(1, 1)

    @pl.when(i == NI - 1)
    def _():
        loss_ref[...] = loss_ref[...] * (BETA / float(ROWS * DIM))


_vq_call = pl.pallas_call(
    _vq_body,
    grid=(NI,),
    in_specs=[
        pl.BlockSpec((RB, DIM), lambda i: (i, 0)),
        pl.BlockSpec((DIM, N_TOK), lambda i: (0, 0)),
    ],
    out_specs=[
        pl.BlockSpec((1, 1, RB), lambda i: (i, 0, 0)),
        pl.BlockSpec((1, 1), lambda i: (0, 0)),
    ],
    out_shape=[
        jax.ShapeDtypeStruct((NI, 1, RB), jnp.int32),
        jax.ShapeDtypeStruct((1, 1), jnp.float32),
    ],
    scratch_shapes=[pltpu.VMEM((1, N_TOK), jnp.float32),
                    pltpu.VMEM((1, N_TOK), jnp.float32)],
)


@functools.cache
def _sc_gather_call():
    @functools.partial(
        pl.kernel,
        mesh=plsc.VectorSubcoreMesh(core_axis_name="c", subcore_axis_name="s"),
        out_type=jax.ShapeDtypeStruct((ROWS, DIM), jnp.float32),
        scratch_types=[
            pltpu.VMEM((_BPW,), jnp.int32),
            pltpu.VMEM((_BPW, DIM), jnp.float32),
            pltpu.SemaphoreType.DMA,
        ],
    )
    def _sc_gather(table_hbm, idx_hbm, out_hbm, idx_v, rows_v, sem):
        wid = lax.axis_index("s") * _SC_NC + lax.axis_index("c")
        base = wid * _BPW
        pltpu.sync_copy(idx_hbm.at[pl.ds(base, _BPW)], idx_v)
        pltpu.async_copy(table_hbm.at[idx_v], rows_v, sem).wait()
        pltpu.sync_copy(rows_v, out_hbm.at[pl.ds(base, _BPW)])

    return _sc_gather


def kernel(z, W):
    zp = jnp.transpose(z, (0, 2, 3, 1))
    z_flat = zp.reshape(ROWS, DIM)
    wt = W.T
    idx3, loss11 = _vq_call(z_flat, wt)
    idx = idx3.reshape(ROWS)
    z_q_rows = _sc_gather_call()(W, idx)
    z_q_out = jnp.transpose(z_q_rows.reshape(zp.shape), (0, 3, 1, 2))
    return (z_q_out, loss11[0, 0], idx)


# speedup vs baseline: 1.1113x; 1.0263x over previous
import functools

import jax
import jax.numpy as jnp
from jax import lax
from jax.experimental import pallas as pl
from jax.experimental.pallas import tpu as pltpu
from jax.experimental.pallas import tpu_sc as plsc

BETA = 0.25
N_TOK = 8192
DIM = 256
ROWS = 8192
BATCH = 8
HW = 1024
BPB = 2
RB = BPB * HW
NI = ROWS // RB
CW = 1024
NCH = N_TOK // CW

_SC_NC = 2
_SC_NS = 16
_SC_NW = _SC_NC * _SC_NS
_BPW = ROWS // _SC_NW


def _vq_body(z_ref, w_ref, idx_ref, loss_ref, wn_ref, rowf_ref):
    i = pl.program_id(0)

    @pl.when(i == 0)
    def _():
        w = w_ref[...]
        wn_ref[...] = jnp.sum(w * w, axis=1, keepdims=True)
        rowf_ref[...] = lax.broadcasted_iota(
            jnp.int32, (N_TOK, 1), 0).astype(jnp.float32)
        loss_ref[...] = jnp.zeros((1, 1), jnp.float32)

    zt = jnp.concatenate([z_ref[b] for b in range(BPB)], axis=1)
    znt = jnp.sum(zt * zt, axis=0, keepdims=True)
    zt2 = zt * (-2.0)
    best_m = best_i = None
    for c in range(NCH):
        sl = pl.ds(c * CW, CW)
        s2t = lax.dot_general(w_ref[sl, :], zt2, (((1,), (0,)), ((), ())),
                              preferred_element_type=jnp.float32)
        dt = (znt + wn_ref[sl, :]) + s2t
        mc = jnp.min(dt, axis=0, keepdims=True)
        ic = jnp.min(jnp.where(dt == mc, rowf_ref[sl, :], float(N_TOK)),
                     axis=0, keepdims=True)
        if c == 0:
            best_m, best_i = mc, ic
        else:
            take = mc < best_m
            best_i = jnp.where(take, ic, best_i)
            best_m = jnp.where(take, mc, best_m)
    idx_ref[0, 0, :] = best_i[0, :].astype(jnp.int32)
    loss_ref[...] += jnp.sum(best_m).reshape(1, 1)

    @pl.when(i == NI - 1)
    def _():
        loss_ref[...] = loss_ref[...] * (BETA / float(ROWS * DIM))


_vq_call = pl.pallas_call(
    _vq_body,
    grid=(NI,),
    in_specs=[
        pl.BlockSpec((BPB, DIM, HW), lambda i: (i, 0, 0)),
        pl.BlockSpec((N_TOK, DIM), lambda i: (0, 0)),
    ],
    out_specs=[
        pl.BlockSpec((1, 1, RB), lambda i: (i, 0, 0)),
        pl.BlockSpec((1, 1), lambda i: (0, 0)),
    ],
    out_shape=[
        jax.ShapeDtypeStruct((NI, 1, RB), jnp.int32),
        jax.ShapeDtypeStruct((1, 1), jnp.float32),
    ],
    scratch_shapes=[pltpu.VMEM((N_TOK, 1), jnp.float32),
                    pltpu.VMEM((N_TOK, 1), jnp.float32)],
)


@functools.cache
def _sc_gather_call():
    @functools.partial(
        pl.kernel,
        mesh=plsc.VectorSubcoreMesh(core_axis_name="c", subcore_axis_name="s"),
        out_type=jax.ShapeDtypeStruct((ROWS, DIM), jnp.float32),
        scratch_types=[
            pltpu.VMEM((_BPW,), jnp.int32),
            pltpu.VMEM((_BPW, DIM), jnp.float32),
            pltpu.SemaphoreType.DMA,
        ],
    )
    def _sc_gather(table_hbm, idx_hbm, out_hbm, idx_v, rows_v, sem):
        wid = lax.axis_index("s") * _SC_NC + lax.axis_index("c")
        base = wid * _BPW
        pltpu.sync_copy(idx_hbm.at[pl.ds(base, _BPW)], idx_v)
        pltpu.async_copy(table_hbm.at[idx_v], rows_v, sem).wait()
        pltpu.sync_copy(rows_v, out_hbm.at[pl.ds(base, _BPW)])

    return _sc_gather


def kernel(z, W):
    z_r = z.reshape(BATCH, DIM, HW)
    idx3, loss11 = _vq_call(z_r, W)
    idx = idx3.reshape(ROWS)
    z_q_rows = _sc_gather_call()(W, idx)
    z_q_out = jnp.transpose(
        z_q_rows.reshape(BATCH, 32, 32, DIM), (0, 3, 1, 2))
    return (z_q_out, loss11[0, 0], idx)


# speedup vs baseline: 1.1134x; 1.0019x over previous
import functools

import jax
import jax.numpy as jnp
from jax import lax
from jax.experimental import pallas as pl
from jax.experimental.pallas import tpu as pltpu
from jax.experimental.pallas import tpu_sc as plsc

BETA = 0.25
N_TOK = 8192
DIM = 256
ROWS = 8192
BATCH = 8
HW = 1024
BPB = 2
RB = BPB * HW
NI = ROWS // RB
CW = 1024
NCH = N_TOK // CW

_SC_NC = 2
_SC_NS = 16
_SC_NW = _SC_NC * _SC_NS
_BPW = ROWS // _SC_NW


def _vq_body(z_ref, w_ref, idx_ref, loss_ref, wn_ref, rowf_ref):
    i = pl.program_id(0)

    @pl.when(i == 0)
    def _():
        w = w_ref[...]
        wn_ref[...] = jnp.sum(w * w, axis=1, keepdims=True)
        rowf_ref[...] = lax.broadcasted_iota(
            jnp.int32, (N_TOK, 1), 0).astype(jnp.float32)
        loss_ref[...] = jnp.zeros((1, 1), jnp.float32)

    zt = jnp.concatenate([z_ref[b] for b in range(BPB)], axis=1)
    znt = jnp.sum(zt * zt, axis=0, keepdims=True)
    zt2 = zt * (-2.0)
    best_m = best_i = None
    for c in range(NCH):
        sl = pl.ds(c * CW, CW)
        s2t = lax.dot_general(w_ref[sl, :], zt2, (((1,), (0,)), ((), ())),
                              preferred_element_type=jnp.float32)
        dt = (znt + wn_ref[sl, :]) + s2t
        mc = jnp.min(dt, axis=0, keepdims=True)
        ic = jnp.min(jnp.where(dt == mc, rowf_ref[sl, :], float(N_TOK)),
                     axis=0, keepdims=True)
        if c == 0:
            best_m, best_i = mc, ic
        else:
            take = mc < best_m
            best_i = jnp.where(take, ic, best_i)
            best_m = jnp.where(take, mc, best_m)
    idx_ref[0, 0, :] = best_i[0, :].astype(jnp.int32)
    loss_ref[...] += jnp.sum(best_m).reshape(1, 1)

    @pl.when(i == NI - 1)
    def _():
        loss_ref[...] = loss_ref[...] * (BETA / float(ROWS * DIM))


_vq_call = pl.pallas_call(
    _vq_body,
    grid=(NI,),
    in_specs=[
        pl.BlockSpec((BPB, DIM, HW), lambda i: (i, 0, 0)),
        pl.BlockSpec((N_TOK, DIM), lambda i: (0, 0)),
    ],
    out_specs=[
        pl.BlockSpec((1, 1, RB), lambda i: (i, 0, 0)),
        pl.BlockSpec((1, 1), lambda i: (0, 0)),
    ],
    out_shape=[
        jax.ShapeDtypeStruct((NI, 1, RB), jnp.int32),
        jax.ShapeDtypeStruct((1, 1), jnp.float32),
    ],
    scratch_shapes=[pltpu.VMEM((N_TOK, 1), jnp.float32),
                    pltpu.VMEM((N_TOK, 1), jnp.float32)],
)


@functools.cache
def _sc_gather_call():
    @functools.partial(
        pl.kernel,
        mesh=plsc.VectorSubcoreMesh(core_axis_name="c", subcore_axis_name="s"),
        compiler_params=pltpu.CompilerParams(use_tc_tiling_on_sc=True),
        out_type=jax.ShapeDtypeStruct((ROWS, DIM), jnp.float32),
        scratch_types=[
            pltpu.VMEM((_BPW,), jnp.int32),
            pltpu.VMEM((_BPW, DIM), jnp.float32),
            pltpu.SemaphoreType.DMA,
        ],
    )
    def _sc_gather(table_hbm, idx_hbm, out_hbm, idx_v, rows_v, sem):
        wid = lax.axis_index("s") * _SC_NC + lax.axis_index("c")
        base = wid * _BPW
        pltpu.sync_copy(idx_hbm.at[pl.ds(base, _BPW)], idx_v)
        pltpu.async_copy(table_hbm.at[idx_v], rows_v, sem).wait()
        pltpu.sync_copy(rows_v, out_hbm.at[pl.ds(base, _BPW)])

    return _sc_gather


def kernel(z, W):
    z_r = z.reshape(BATCH, DIM, HW)
    idx3, loss11 = _vq_call(z_r, W)
    idx = idx3.reshape(ROWS)
    z_q_rows = _sc_gather_call()(W, idx)
    z_q_out = jnp.transpose(
        z_q_rows.reshape(BATCH, 32, 32, DIM), (0, 3, 1, 2))
    return (z_q_out, loss11[0, 0], idx)
